# baseline (speedup 1.0000x reference)
import functools

import jax
import jax.numpy as jnp
from jax import lax
from jax.experimental import pallas as pl
from jax.experimental.pallas import tpu as pltpu
from jax.experimental.pallas import tpu_sc as plsc

N = 10000
D = 128
E = 320000

NC = 2
NS = 16
NW = NC * NS
NPT = 320
NPAD = NW * NPT
CHUNK = 4000
NCHUNKS = E // CHUNK
K = 64
RING_B = 6
RING_R = 8
GL = 2
SPILL_BLK = 2048
TRASH = 4096
CPK_CAP = TRASH + 16
LCAP = E + 2048
DUMMY_PK = NPT


def _wid_lo():
    wid = lax.axis_index("s") * NC + lax.axis_index("c")
    return wid, wid * NPT


def _init_acc(acc):
    neg = jnp.full((16,), -jnp.inf, dtype=jnp.float32)

    def init_row(i, _):
        r = i // 8
        f = i % 8
        acc[r, pl.ds(f * 16, 16)] = neg
        return 0

    lax.fori_loop(0, (NPT + 1) * 8, init_row, 0, unroll=8)


def _sc_phase2(y_hbm, lists_hbm, lbase, nu, acc, pbuf, idxb, dstu, rows,
               lsem, gsem, ring):
    pb = [pbuf.at[j] for j in range(ring)]
    ib = [idxb.at[j] for j in range(ring)]
    db = [dstu.at[j] for j in range(ring)]
    rb = [rows.at[j] for j in range(ring)]
    ml = ring - 1

    def macro(ms, _):
        for j in range(ring):
            t = ms * ring + j

            @pl.when(t < nu)
            def _():
                pltpu.async_copy(lists_hbm.at[pl.ds(lbase + t * K, K)],
                                 pb[j], lsem.at[j])

            t2 = t - GL
            s2 = (j - GL) % ring

            @pl.when((t2 >= 0) & (t2 < nu))
            def _():
                pltpu.make_async_copy(
                    lists_hbm.at[pl.ds(lbase + t2 * K, K)],
                    pb[s2], lsem.at[s2]).wait()
                for r in range(K // 16):
                    v = pb[s2][pl.ds(r * 16, 16)]
                    ib[s2][pl.ds(r * 16, 16)] = v >> 9
                    db[s2][pl.ds(r * 16, 16)] = v & 511
                pltpu.async_copy(y_hbm.at[ib[s2]], rb[s2], gsem.at[s2])

            t3 = t - ml
            s3 = (j - ml) % ring

            @pl.when((t3 >= 0) & (t3 < nu))
            def _():
                pltpu.make_async_copy(y_hbm.at[ib[s3]], rb[s3],
                                      gsem.at[s3]).wait()

                def group(g, _):
                    dv = db[s3][pl.ds(g * 16, 16)]
                    for lane in range(16):
                        dl = dv[lane]
                        jr = g * 16 + lane
                        for f in range(8):
                            sl = pl.ds(f * 16, 16)
                            acc[dl, sl] = jnp.maximum(acc[dl, sl],
                                                      rb[s3][jr, sl])
                    return 0

                lax.fori_loop(0, K // 16, group, 0)
        return 0

    lax.fori_loop(0, (nu + ml + ring) // ring, macro, 0)


def _sc_build_body(y_hbm, src_hbm, dst_hbm,
                   out_hbm, lists_hbm, totals_hbm,
                   acc, srcb0, srcb1, dstb0, dstb1, cpk0, cpk1,
                   pbuf, idxb, dstu, rows, dbuf, tb,
                   lsem, gsem, csem, ssem):
    srcb = [srcb0, srcb1]
    dstb = [dstb0, dstb1]
    cpk = [cpk0, cpk1]
    wid, lo = _wid_lo()
    _init_acc(acc)
    lbase = wid * LCAP

    dummy_pk = jnp.full((16,), DUMMY_PK, jnp.int32)
    lanes = lax.iota(jnp.int32, 16)
    for r in range(8):
        dbuf[pl.ds(r * 16, 16)] = dummy_pk

    def start_chunk(c, p):
        base = c * CHUNK
        pltpu.async_copy(src_hbm.at[pl.ds(base, CHUNK)], srcb[p],
                         csem.at[p])
        pltpu.async_copy(dst_hbm.at[pl.ds(base, CHUNK)], dstb[p],
                         csem.at[p])

    def wait_chunk(c, p):
        base = c * CHUNK
        pltpu.make_async_copy(src_hbm.at[pl.ds(base, CHUNK)], srcb[p],
                              csem.at[p]).wait()
        pltpu.make_async_copy(dst_hbm.at[pl.ds(base, CHUNK)], dstb[p],
                              csem.at[p]).wait()

    def drain_spill(p, nblk):
        def w(b, _):
            pltpu.make_async_copy(
                cpk[p].at[pl.ds(b * SPILL_BLK, SPILL_BLK)],
                lists_hbm.at[pl.ds(lbase, SPILL_BLK)],
                ssem.at[p]).wait()
            return 0
        lax.fori_loop(0, nblk, w, 0)

    start_chunk(0, 0)

    def chunk_pair(cc, carry):
        cursor, n0, n1 = carry
        nprev = [n0, n1]
        for p in range(2):
            c = cc * 2 + p
            drain_spill(p, nprev[p])
            wait_chunk(c, p)

            @pl.when(c + 1 < NCHUNKS)
            def _():
                start_chunk(c + 1, 1 - p)

            def filt(i, cnt):
                dv = dstb[p][pl.ds(i * 16, 16)]
                sv = srcb[p][pl.ds(i * 16, 16)]
                rel = dv - lo
                m = (rel >= 0) & (rel < NPT)
                incl = plsc.cumsum(jnp.where(m, 1, 0))
                pos = jnp.where(m, cnt + incl - 1, TRASH + lanes)
                plsc.store_scatter(cpk[p], [pos],
                                   (sv << 9) | (rel & 511))
                pc = plsc.all_reduce_population_count(m)
                return cnt + pc[0]

            cnt = lax.fori_loop(0, CHUNK // 16, filt, 0, unroll=4)
            cpk[p][pl.ds(cnt, 16)] = dummy_pk
            r8 = ((cnt + 7) // 8) * 8
            nblk = (r8 + SPILL_BLK - 1) // SPILL_BLK

            def spill(b, _):
                off = pl.multiple_of(lbase + cursor + b * SPILL_BLK, 8)
                pltpu.async_copy(
                    cpk[p].at[pl.ds(b * SPILL_BLK, SPILL_BLK)],
                    lists_hbm.at[pl.ds(off, SPILL_BLK)],
                    ssem.at[p])
                return 0

            lax.fori_loop(0, nblk, spill, 0)
            cursor = cursor + r8
            nprev[p] = nblk
        return (cursor, nprev[0], nprev[1])

    cursor, n0, n1 = lax.fori_loop(0, NCHUNKS // 2, chunk_pair, (0, 0, 0))
    drain_spill(0, n0)
    drain_spill(1, n1)

    pltpu.sync_copy(
        dbuf, lists_hbm.at[pl.ds(pl.multiple_of(lbase + cursor, 8), 128)])
    nu = (cursor + K - 1) // K
    tb[pl.ds(0, 16)] = jnp.zeros((16,), jnp.int32) + nu
    pltpu.sync_copy(tb, totals_hbm.at[pl.ds(wid * 16, 16)])

    _sc_phase2(y_hbm, lists_hbm, lbase, nu, acc, pbuf, idxb, dstu, rows,
               lsem, gsem, RING_B)

    pltpu.sync_copy(acc.at[pl.ds(0, NPT)], out_hbm.at[pl.ds(lo, NPT)])


def _sc_reuse_body(y_hbm, lists_hbm, totals_hbm,
                   out_hbm,
                   acc, pbuf, idxb, dstu, rows, tb,
                   lsem, gsem):
    wid, lo = _wid_lo()
    _init_acc(acc)
    lbase = wid * LCAP

    pltpu.sync_copy(totals_hbm.at[pl.ds(wid * 16, 16)], tb)
    nu = tb[pl.ds(0, 16)][0]

    _sc_phase2(y_hbm, lists_hbm, lbase, nu, acc, pbuf, idxb, dstu, rows,
               lsem, gsem, RING_R)

    pltpu.sync_copy(acc.at[pl.ds(0, NPT)], out_hbm.at[pl.ds(lo, NPT)])


_SC_MESH = plsc.VectorSubcoreMesh(core_axis_name="c", subcore_axis_name="s")


def _ring_scratch(ring):
    return [
        pltpu.VMEM((ring, K), jnp.int32),
        pltpu.VMEM((ring, K), jnp.int32),
        pltpu.VMEM((ring, K), jnp.int32),
        pltpu.VMEM((ring, K, D), jnp.float32),
    ]


_sc_build = functools.partial(
    pl.kernel,
    out_type=(jax.ShapeDtypeStruct((NPAD, D), jnp.float32),
              jax.ShapeDtypeStruct((NW * LCAP,), jnp.int32),
              jax.ShapeDtypeStruct((NW * 16,), jnp.int32)),
    mesh=_SC_MESH,
    compiler_params=pltpu.CompilerParams(needs_layout_passes=False),
    scratch_types=[
        pltpu.VMEM((NPT + 1, D), jnp.float32),
        pltpu.VMEM((CHUNK,), jnp.int32),
        pltpu.VMEM((CHUNK,), jnp.int32),
        pltpu.VMEM((CHUNK,), jnp.int32),
        pltpu.VMEM((CHUNK,), jnp.int32),
        pltpu.VMEM((CPK_CAP,), jnp.int32),
        pltpu.VMEM((CPK_CAP,), jnp.int32),
    ] + _ring_scratch(RING_B) + [
        pltpu.VMEM((128,), jnp.int32),
        pltpu.VMEM((16,), jnp.int32),
        pltpu.SemaphoreType.DMA((RING_B,)),
        pltpu.SemaphoreType.DMA((RING_B,)),
        pltpu.SemaphoreType.DMA((2,)),
        pltpu.SemaphoreType.DMA((2,)),
    ],
)(_sc_build_body)

_sc_reuse = functools.partial(
    pl.kernel,
    out_type=jax.ShapeDtypeStruct((NPAD, D), jnp.float32),
    mesh=_SC_MESH,
    compiler_params=pltpu.CompilerParams(needs_layout_passes=False),
    scratch_types=[
        pltpu.VMEM((NPT + 1, D), jnp.float32),
    ] + _ring_scratch(RING_R) + [
        pltpu.VMEM((16,), jnp.int32),
        pltpu.SemaphoreType.DMA((RING_R,)),
        pltpu.SemaphoreType.DMA((RING_R,)),
    ],
)(_sc_reuse_body)


def _shifts(v):
    z = jnp.zeros((v.shape[0], 1), v.dtype)
    vl = jnp.concatenate([z, v[:, :-1]], axis=1)
    vr = jnp.concatenate([v[:, 1:], z], axis=1)
    return vl, vr


def _conv3(v, w, b):
    vl, vr = _shifts(v)
    return w[0] * vl + w[1] * v + w[2] * vr + b


def _conv3x2(v, a, w, b):
    vl, vr = _shifts(v)
    al, ar = _shifts(a)
    return (w[0] * vl + w[1] * v + w[2] * vr
            + w[3] * al + w[4] * a + w[5] * ar + b)


def _tc_pre_body(x_ref, w_ref, b_ref, y_ref):
    y_ref[...] = _conv3(x_ref[...], w_ref, b_ref[0])


def _tc_mid_body(x_ref, agg_ref, uw_ref, ub_ref, mw_ref, mb_ref,
                 h_ref, y_ref):
    a = agg_ref[...]
    a = jnp.where(jnp.isneginf(a), 0.0, a)
    h = jnp.maximum(_conv3x2(x_ref[...], a, uw_ref, ub_ref[0]), 0.0)
    h_ref[...] = h
    y_ref[...] = _conv3(h, mw_ref, mb_ref[0])


def _tc_final_body(h_ref, agg_ref, uw_ref, ub_ref, wt_ref, bp_ref, o_ref):
    a = agg_ref[...]
    a = jnp.where(jnp.isneginf(a), 0.0, a)
    h2 = jnp.maximum(_conv3x2(h_ref[...], a, uw_ref, ub_ref[0]), 0.0)
    m = jnp.max(h2, axis=1, keepdims=True)
    o_ref[...] = (jnp.sum(m * wt_ref[...], axis=0, keepdims=True)
                  + bp_ref[...])


_smem_spec = pl.BlockSpec(memory_space=pltpu.SMEM)
_vmem_spec = pl.BlockSpec(memory_space=pltpu.VMEM)

_tc_pre = pl.pallas_call(
    _tc_pre_body,
    out_shape=jax.ShapeDtypeStruct((N, D), jnp.float32),
    in_specs=[_vmem_spec, _smem_spec, _smem_spec],
    out_specs=_vmem_spec,
)

_tc_mid = pl.pallas_call(
    _tc_mid_body,
    out_shape=(jax.ShapeDtypeStruct((N, D), jnp.float32),
               jax.ShapeDtypeStruct((N, D), jnp.float32)),
    in_specs=[_vmem_spec, _vmem_spec, _smem_spec, _smem_spec,
              _smem_spec, _smem_spec],
    out_specs=(_vmem_spec, _vmem_spec),
)

_tc_final = pl.pallas_call(
    _tc_final_body,
    out_shape=jax.ShapeDtypeStruct((1, D), jnp.float32),
    in_specs=[_vmem_spec, _vmem_spec, _smem_spec, _smem_spec,
              _vmem_spec, _vmem_spec],
    out_specs=_vmem_spec,
)


def kernel(x, edge_index, mf_w0, mf_b0, uf_w0, uf_b0,
           mf_w1, mf_b1, uf_w1, uf_b1, W_out, b_out):
    src = edge_index[0]
    dst = edge_index[1]
    mw0 = mf_w0.reshape(3)
    uw0 = uf_w0.reshape(6)
    mw1 = mf_w1.reshape(3)
    uw1 = uf_w1.reshape(6)
    wt = jnp.pad(W_out.T, ((0, 0), (0, D - W_out.shape[0])))
    bp = jnp.pad(b_out, (0, D - b_out.shape[0]))[None, :]

    y0 = _tc_pre(x, mw0, mf_b0)
    agg0, lists, totals = _sc_build(y0, src, dst)
    h1, y1 = _tc_mid(x, agg0[:N], uw0, uf_b0, mw1, mf_b1)
    agg1 = _sc_reuse(y1, lists, totals)
    res = _tc_final(h1, agg1[:N], uw1, uf_b1, wt, bp)
    return res[:, :3]


# speedup vs baseline: 1.0180x; 1.0180x over previous
import functools

import jax
import jax.numpy as jnp
from jax import lax
from jax.experimental import pallas as pl
from jax.experimental.pallas import tpu as pltpu
from jax.experimental.pallas import tpu_sc as plsc

N = 10000
D = 128
E = 320000

NC = 2
NS = 16
NW = NC * NS
NPT = 320
NPAD = NW * NPT
CHUNK = 4000
NCHUNKS = E // CHUNK
K = 64
KR = 128
RING_B = 4
RING_R = 4
GL = 2
SPILL_BLK = 2048
TRASH = 4096
CPK_CAP = TRASH + 16
LCAP = E + 2048
DUMMY_PK = NPT


def _wid_lo():
    wid = lax.axis_index("s") * NC + lax.axis_index("c")
    return wid, wid * NPT


def _init_acc(acc):
    neg = jnp.full((16,), -jnp.inf, dtype=jnp.float32)

    def init_row(i, _):
        r = i // 8
        f = i % 8
        acc[r, pl.ds(f * 16, 16)] = neg
        return 0

    lax.fori_loop(0, (NPT + 1) * 8, init_row, 0, unroll=8)


def _sc_phase2(y_hbm, lists_hbm, lbase, nu, acc, pbuf, idxb, dstu, rows,
               lsem, gsem, ring, k):
    pb = [pbuf.at[j] for j in range(ring)]
    ib = [idxb.at[j] for j in range(ring)]
    db = [dstu.at[j] for j in range(ring)]
    rb = [rows.at[j] for j in range(ring)]
    ml = ring - 1

    def macro(ms, _):
        for j in range(ring):
            t = ms * ring + j

            @pl.when(t < nu)
            def _():
                pltpu.async_copy(lists_hbm.at[pl.ds(lbase + t * k, k)],
                                 pb[j], lsem.at[j])

            t2 = t - GL
            s2 = (j - GL) % ring

            @pl.when((t2 >= 0) & (t2 < nu))
            def _():
                pltpu.make_async_copy(
                    lists_hbm.at[pl.ds(lbase + t2 * k, k)],
                    pb[s2], lsem.at[s2]).wait()
                for r in range(k // 16):
                    v = pb[s2][pl.ds(r * 16, 16)]
                    ib[s2][pl.ds(r * 16, 16)] = v >> 9
                    db[s2][pl.ds(r * 16, 16)] = v & 511
                pltpu.async_copy(y_hbm.at[ib[s2]], rb[s2], gsem.at[s2])

            t3 = t - ml
            s3 = (j - ml) % ring

            @pl.when((t3 >= 0) & (t3 < nu))
            def _():
                pltpu.make_async_copy(y_hbm.at[ib[s3]], rb[s3],
                                      gsem.at[s3]).wait()

                def group(g, _):
                    dv = db[s3][pl.ds(g * 16, 16)]
                    for lane in range(16):
                        dl = dv[lane]
                        jr = g * 16 + lane
                        for f in range(8):
                            sl = pl.ds(f * 16, 16)
                            acc[dl, sl] = jnp.maximum(acc[dl, sl],
                                                      rb[s3][jr, sl])
                    return 0

                lax.fori_loop(0, k // 16, group, 0)
        return 0

    lax.fori_loop(0, (nu + ml + ring) // ring, macro, 0)


def _sc_build_body(y_hbm, src_hbm, dst_hbm,
                   out_hbm, lists_hbm, totals_hbm,
                   acc, srcb0, srcb1, dstb0, dstb1, cpk0, cpk1,
                   pbuf, idxb, dstu, rows, dbuf, tb,
                   lsem, gsem, csem, ssem):
    srcb = [srcb0, srcb1]
    dstb = [dstb0, dstb1]
    cpk = [cpk0, cpk1]
    wid, lo = _wid_lo()
    _init_acc(acc)
    lbase = wid * LCAP

    dummy_pk = jnp.full((16,), DUMMY_PK, jnp.int32)
    lanes = lax.iota(jnp.int32, 16)
    for r in range(8):
        dbuf[pl.ds(r * 16, 16)] = dummy_pk

    def start_chunk(c, p):
        base = c * CHUNK
        pltpu.async_copy(src_hbm.at[pl.ds(base, CHUNK)], srcb[p],
                         csem.at[p])
        pltpu.async_copy(dst_hbm.at[pl.ds(base, CHUNK)], dstb[p],
                         csem.at[p])

    def wait_chunk(c, p):
        base = c * CHUNK
        pltpu.make_async_copy(src_hbm.at[pl.ds(base, CHUNK)], srcb[p],
                              csem.at[p]).wait()
        pltpu.make_async_copy(dst_hbm.at[pl.ds(base, CHUNK)], dstb[p],
                              csem.at[p]).wait()

    def drain_spill(p, nblk):
        def w(b, _):
            pltpu.make_async_copy(
                cpk[p].at[pl.ds(b * SPILL_BLK, SPILL_BLK)],
                lists_hbm.at[pl.ds(lbase, SPILL_BLK)],
                ssem.at[p]).wait()
            return 0
        lax.fori_loop(0, nblk, w, 0)

    start_chunk(0, 0)

    def chunk_pair(cc, carry):
        cursor, n0, n1 = carry
        nprev = [n0, n1]
        for p in range(2):
            c = cc * 2 + p
            drain_spill(p, nprev[p])
            wait_chunk(c, p)

            @pl.when(c + 1 < NCHUNKS)
            def _():
                start_chunk(c + 1, 1 - p)

            def filt(i, cnt):
                dv = dstb[p][pl.ds(i * 16, 16)]
                sv = srcb[p][pl.ds(i * 16, 16)]
                rel = dv - lo
                m = (rel >= 0) & (rel < NPT)
                incl = plsc.cumsum(jnp.where(m, 1, 0))
                pos = jnp.where(m, cnt + incl - 1, TRASH + lanes)
                plsc.store_scatter(cpk[p], [pos],
                                   (sv << 9) | (rel & 511))
                pc = plsc.all_reduce_population_count(m)
                return cnt + pc[0]

            cnt = lax.fori_loop(0, CHUNK // 16, filt, 0, unroll=4)
            cpk[p][pl.ds(cnt, 16)] = dummy_pk
            r8 = ((cnt + 7) // 8) * 8
            nblk = (r8 + SPILL_BLK - 1) // SPILL_BLK

            def spill(b, _):
                off = pl.multiple_of(lbase + cursor + b * SPILL_BLK, 8)
                pltpu.async_copy(
                    cpk[p].at[pl.ds(b * SPILL_BLK, SPILL_BLK)],
                    lists_hbm.at[pl.ds(off, SPILL_BLK)],
                    ssem.at[p])
                return 0

            lax.fori_loop(0, nblk, spill, 0)
            cursor = cursor + r8
            nprev[p] = nblk
        return (cursor, nprev[0], nprev[1])

    cursor, n0, n1 = lax.fori_loop(0, NCHUNKS // 2, chunk_pair, (0, 0, 0))
    drain_spill(0, n0)
    drain_spill(1, n1)

    pltpu.sync_copy(
        dbuf, lists_hbm.at[pl.ds(pl.multiple_of(lbase + cursor, 8), KR)])
    tb[pl.ds(0, 16)] = jnp.zeros((16,), jnp.int32) + cursor
    pltpu.sync_copy(tb, totals_hbm.at[pl.ds(wid * 16, 16)])

    nu = (cursor + K - 1) // K
    _sc_phase2(y_hbm, lists_hbm, lbase, nu, acc, pbuf, idxb, dstu, rows,
               lsem, gsem, RING_B, K)

    pltpu.sync_copy(acc.at[pl.ds(0, NPT)], out_hbm.at[pl.ds(lo, NPT)])


def _sc_reuse_body(y_hbm, lists_hbm, totals_hbm,
                   out_hbm,
                   acc, pbuf, idxb, dstu, rows, tb,
                   lsem, gsem):
    wid, lo = _wid_lo()
    _init_acc(acc)
    lbase = wid * LCAP

    pltpu.sync_copy(totals_hbm.at[pl.ds(wid * 16, 16)], tb)
    cursor = tb[pl.ds(0, 16)][0]
    nu = (cursor + KR - 1) // KR

    _sc_phase2(y_hbm, lists_hbm, lbase, nu, acc, pbuf, idxb, dstu, rows,
               lsem, gsem, RING_R, KR)

    pltpu.sync_copy(acc.at[pl.ds(0, NPT)], out_hbm.at[pl.ds(lo, NPT)])


_SC_MESH = plsc.VectorSubcoreMesh(core_axis_name="c", subcore_axis_name="s")


def _ring_scratch(ring, k):
    return [
        pltpu.VMEM((ring, k), jnp.int32),
        pltpu.VMEM((ring, k), jnp.int32),
        pltpu.VMEM((ring, k), jnp.int32),
        pltpu.VMEM((ring, k, D), jnp.float32),
    ]


_sc_build = functools.partial(
    pl.kernel,
    out_type=(jax.ShapeDtypeStruct((NPAD, D), jnp.float32),
              jax.ShapeDtypeStruct((NW * LCAP,), jnp.int32),
              jax.ShapeDtypeStruct((NW * 16,), jnp.int32)),
    mesh=_SC_MESH,
    compiler_params=pltpu.CompilerParams(needs_layout_passes=False),
    scratch_types=[
        pltpu.VMEM((NPT + 1, D), jnp.float32),
        pltpu.VMEM((CHUNK,), jnp.int32),
        pltpu.VMEM((CHUNK,), jnp.int32),
        pltpu.VMEM((CHUNK,), jnp.int32),
        pltpu.VMEM((CHUNK,), jnp.int32),
        pltpu.VMEM((CPK_CAP,), jnp.int32),
        pltpu.VMEM((CPK_CAP,), jnp.int32),
    ] + _ring_scratch(RING_B, K) + [
        pltpu.VMEM((KR,), jnp.int32),
        pltpu.VMEM((16,), jnp.int32),
        pltpu.SemaphoreType.DMA((RING_B,)),
        pltpu.SemaphoreType.DMA((RING_B,)),
        pltpu.SemaphoreType.DMA((2,)),
        pltpu.SemaphoreType.DMA((2,)),
    ],
)(_sc_build_body)

_sc_reuse = functools.partial(
    pl.kernel,
    out_type=jax.ShapeDtypeStruct((NPAD, D), jnp.float32),
    mesh=_SC_MESH,
    compiler_params=pltpu.CompilerParams(needs_layout_passes=False),
    scratch_types=[
        pltpu.VMEM((NPT + 1, D), jnp.float32),
    ] + _ring_scratch(RING_R, KR) + [
        pltpu.VMEM((16,), jnp.int32),
        pltpu.SemaphoreType.DMA((RING_R,)),
        pltpu.SemaphoreType.DMA((RING_R,)),
    ],
)(_sc_reuse_body)


def _shifts(v):
    z = jnp.zeros((v.shape[0], 1), v.dtype)
    vl = jnp.concatenate([z, v[:, :-1]], axis=1)
    vr = jnp.concatenate([v[:, 1:], z], axis=1)
    return vl, vr


def _conv3(v, w, b):
    vl, vr = _shifts(v)
    return w[0] * vl + w[1] * v + w[2] * vr + b


def _conv3x2(v, a, w, b):
    vl, vr = _shifts(v)
    al, ar = _shifts(a)
    return (w[0] * vl + w[1] * v + w[2] * vr
            + w[3] * al + w[4] * a + w[5] * ar + b)


def _tc_pre_body(x_ref, w_ref, b_ref, y_ref):
    y_ref[...] = _conv3(x_ref[...], w_ref, b_ref[0])


def _tc_mid_body(x_ref, agg_ref, uw_ref, ub_ref, mw_ref, mb_ref,
                 h_ref, y_ref):
    a = agg_ref[...]
    a = jnp.where(jnp.isneginf(a), 0.0, a)
    h = jnp.maximum(_conv3x2(x_ref[...], a, uw_ref, ub_ref[0]), 0.0)
    h_ref[...] = h
    y_ref[...] = _conv3(h, mw_ref, mb_ref[0])


def _tc_final_body(h_ref, agg_ref, uw_ref, ub_ref, wt_ref, bp_ref, o_ref):
    a = agg_ref[...]
    a = jnp.where(jnp.isneginf(a), 0.0, a)
    h2 = jnp.maximum(_conv3x2(h_ref[...], a, uw_ref, ub_ref[0]), 0.0)
    m = jnp.max(h2, axis=1, keepdims=True)
    o_ref[...] = (jnp.sum(m * wt_ref[...], axis=0, keepdims=True)
                  + bp_ref[...])


_smem_spec = pl.BlockSpec(memory_space=pltpu.SMEM)
_vmem_spec = pl.BlockSpec(memory_space=pltpu.VMEM)

_tc_pre = pl.pallas_call(
    _tc_pre_body,
    out_shape=jax.ShapeDtypeStruct((N, D), jnp.float32),
    in_specs=[_vmem_spec, _smem_spec, _smem_spec],
    out_specs=_vmem_spec,
)

_tc_mid = pl.pallas_call(
    _tc_mid_body,
    out_shape=(jax.ShapeDtypeStruct((N, D), jnp.float32),
               jax.ShapeDtypeStruct((N, D), jnp.float32)),
    in_specs=[_vmem_spec, _vmem_spec, _smem_spec, _smem_spec,
              _smem_spec, _smem_spec],
    out_specs=(_vmem_spec, _vmem_spec),
)

_tc_final = pl.pallas_call(
    _tc_final_body,
    out_shape=jax.ShapeDtypeStruct((1, D), jnp.float32),
    in_specs=[_vmem_spec, _vmem_spec, _smem_spec, _smem_spec,
              _vmem_spec, _vmem_spec],
    out_specs=_vmem_spec,
)


def kernel(x, edge_index, mf_w0, mf_b0, uf_w0, uf_b0,
           mf_w1, mf_b1, uf_w1, uf_b1, W_out, b_out):
    src = edge_index[0]
    dst = edge_index[1]
    mw0 = mf_w0.reshape(3)
    uw0 = uf_w0.reshape(6)
    mw1 = mf_w1.reshape(3)
    uw1 = uf_w1.reshape(6)
    wt = jnp.pad(W_out.T, ((0, 0), (0, D - W_out.shape[0])))
    bp = jnp.pad(b_out, (0, D - b_out.shape[0]))[None, :]

    y0 = _tc_pre(x, mw0, mf_b0)
    agg0, lists, totals = _sc_build(y0, src, dst)
    h1, y1 = _tc_mid(x, agg0[:N], uw0, uf_b0, mw1, mf_b1)
    agg1 = _sc_reuse(y1, lists, totals)
    res = _tc_final(h1, agg1[:N], uw1, uf_b1, wt, bp)
    return res[:, :3]


# speedup vs baseline: 1.0346x; 1.0163x over previous
import functools

import jax
import jax.numpy as jnp
from jax import lax
from jax.experimental import pallas as pl
from jax.experimental.pallas import tpu as pltpu
from jax.experimental.pallas import tpu_sc as plsc

N = 10000
D = 128
E = 320000

NC = 2
NS = 16
NW = NC * NS
NPT = 320
NPAD = NW * NPT
CHUNK = 4000
NCHUNKS = E // CHUNK
K = 64
KR = 96
RING_B = 4
RING_R = 4
GL = 2
SPILL_BLK = 2048
TRASH = 4096
CPK_CAP = TRASH + 16
LCAP = E + 2048
DUMMY_PK = NPT


def _wid_lo():
    wid = lax.axis_index("s") * NC + lax.axis_index("c")
    return wid, wid * NPT


def _init_acc(acc):
    neg = jnp.full((16,), -jnp.inf, dtype=jnp.float32)

    def init_row(i, _):
        r = i // 8
        f = i % 8
        acc[r, pl.ds(f * 16, 16)] = neg
        return 0

    lax.fori_loop(0, (NPT + 1) * 8, init_row, 0, unroll=8)


def _sc_phase2(y_hbm, lists_hbm, lbase, nu, acc, pbuf, idxb, dstu, rows,
               lsem, gsem, ring, k):
    pb = [pbuf.at[j] for j in range(ring)]
    ib = [idxb.at[j] for j in range(ring)]
    db = [dstu.at[j] for j in range(ring)]
    rb = [rows.at[j] for j in range(ring)]
    ml = ring - 1

    def macro(ms, _):
        for j in range(ring):
            t = ms * ring + j

            @pl.when(t < nu)
            def _():
                pltpu.async_copy(lists_hbm.at[pl.ds(lbase + t * k, k)],
                                 pb[j], lsem[j])

            t2 = t - GL
            s2 = (j - GL) % ring

            @pl.when((t2 >= 0) & (t2 < nu))
            def _():
                pltpu.make_async_copy(
                    lists_hbm.at[pl.ds(lbase + t2 * k, k)],
                    pb[s2], lsem[s2]).wait()
                for r in range(k // 16):
                    v = pb[s2][pl.ds(r * 16, 16)]
                    ib[s2][pl.ds(r * 16, 16)] = v >> 9
                    db[s2][pl.ds(r * 16, 16)] = v & 511
                pltpu.async_copy(y_hbm.at[ib[s2]], rb[s2], gsem[s2])

            t3 = t - ml
            s3 = (j - ml) % ring

            @pl.when((t3 >= 0) & (t3 < nu))
            def _():
                pltpu.make_async_copy(y_hbm.at[ib[s3]], rb[s3],
                                      gsem[s3]).wait()

                def group(g, _):
                    dv = db[s3][pl.ds(g * 16, 16)]
                    for lane in range(16):
                        dl = dv[lane]
                        jr = g * 16 + lane
                        for f in range(8):
                            sl = pl.ds(f * 16, 16)
                            acc[dl, sl] = jnp.maximum(acc[dl, sl],
                                                      rb[s3][jr, sl])
                    return 0

                lax.fori_loop(0, k // 16, group, 0)
        return 0

    lax.fori_loop(0, (nu + ml + ring) // ring, macro, 0)


def _sc_build_body(y_hbm, src_hbm, dst_hbm,
                   out_hbm, lists_hbm, totals_hbm,
                   acc, srcb0, srcb1, dstb0, dstb1, cpk0, cpk1,
                   pbuf, idxb, dstu, rows, dbuf, tb,
                   ls0, ls1, ls2, ls3, gs0, gs1, gs2, gs3,
                   cs0, cs1, ss0, ss1):
    srcb = [srcb0, srcb1]
    dstb = [dstb0, dstb1]
    cpk = [cpk0, cpk1]
    lsem = [ls0, ls1, ls2, ls3]
    gsem = [gs0, gs1, gs2, gs3]
    csem = [cs0, cs1]
    ssem = [ss0, ss1]
    wid, lo = _wid_lo()
    _init_acc(acc)
    lbase = wid * LCAP

    dummy_pk = jnp.full((16,), DUMMY_PK, jnp.int32)
    lanes = lax.iota(jnp.int32, 16)
    for r in range(KR // 16):
        dbuf[pl.ds(r * 16, 16)] = dummy_pk

    def start_chunk(c, p):
        base = c * CHUNK
        pltpu.async_copy(src_hbm.at[pl.ds(base, CHUNK)], srcb[p],
                         csem[p])
        pltpu.async_copy(dst_hbm.at[pl.ds(base, CHUNK)], dstb[p],
                         csem[p])

    def wait_chunk(c, p):
        base = c * CHUNK
        pltpu.make_async_copy(src_hbm.at[pl.ds(base, CHUNK)], srcb[p],
                              csem[p]).wait()
        pltpu.make_async_copy(dst_hbm.at[pl.ds(base, CHUNK)], dstb[p],
                              csem[p]).wait()

    def drain_spill(p, nblk):
        def w(b, _):
            pltpu.make_async_copy(
                cpk[p].at[pl.ds(b * SPILL_BLK, SPILL_BLK)],
                lists_hbm.at[pl.ds(lbase, SPILL_BLK)],
                ssem[p]).wait()
            return 0
        lax.fori_loop(0, nblk, w, 0)

    start_chunk(0, 0)

    def chunk_pair(cc, carry):
        cursor, n0, n1 = carry
        nprev = [n0, n1]
        for p in range(2):
            c = cc * 2 + p
            drain_spill(p, nprev[p])
            wait_chunk(c, p)

            @pl.when(c + 1 < NCHUNKS)
            def _():
                start_chunk(c + 1, 1 - p)

            def filt(i, cnt):
                dv = dstb[p][pl.ds(i * 16, 16)]
                sv = srcb[p][pl.ds(i * 16, 16)]
                rel = dv - lo
                m = (rel >= 0) & (rel < NPT)
                incl = plsc.cumsum(jnp.where(m, 1, 0))
                pos = jnp.where(m, cnt + incl - 1, TRASH + lanes)
                plsc.store_scatter(cpk[p], [pos],
                                   (sv << 9) | (rel & 511))
                pc = plsc.all_reduce_population_count(m)
                return cnt + pc[0]

            cnt = lax.fori_loop(0, CHUNK // 16, filt, 0, unroll=4)
            cpk[p][pl.ds(cnt, 16)] = dummy_pk
            r8 = ((cnt + 7) // 8) * 8
            nblk = (r8 + SPILL_BLK - 1) // SPILL_BLK

            def spill(b, _):
                off = pl.multiple_of(lbase + cursor + b * SPILL_BLK, 8)
                pltpu.async_copy(
                    cpk[p].at[pl.ds(b * SPILL_BLK, SPILL_BLK)],
                    lists_hbm.at[pl.ds(off, SPILL_BLK)],
                    ssem[p])
                return 0

            lax.fori_loop(0, nblk, spill, 0)
            cursor = cursor + r8
            nprev[p] = nblk
        return (cursor, nprev[0], nprev[1])

    cursor, n0, n1 = lax.fori_loop(0, NCHUNKS // 2, chunk_pair, (0, 0, 0))
    drain_spill(0, n0)
    drain_spill(1, n1)

    pltpu.sync_copy(
        dbuf, lists_hbm.at[pl.ds(pl.multiple_of(lbase + cursor, 8), KR)])
    tb[pl.ds(0, 16)] = jnp.zeros((16,), jnp.int32) + cursor
    pltpu.sync_copy(tb, totals_hbm.at[pl.ds(wid * 16, 16)])

    nu = (cursor + K - 1) // K
    _sc_phase2(y_hbm, lists_hbm, lbase, nu, acc, pbuf, idxb, dstu, rows,
               lsem, gsem, RING_B, K)

    pltpu.sync_copy(acc.at[pl.ds(0, NPT)], out_hbm.at[pl.ds(lo, NPT)])


def _sc_reuse_body(y_hbm, lists_hbm, totals_hbm,
                   out_hbm,
                   acc, pbuf, idxb, dstu, rows, tb,
                   ls0, ls1, ls2, ls3, gs0, gs1, gs2, gs3):
    lsem = [ls0, ls1, ls2, ls3]
    gsem = [gs0, gs1, gs2, gs3]
    wid, lo = _wid_lo()
    _init_acc(acc)
    lbase = wid * LCAP

    pltpu.sync_copy(totals_hbm.at[pl.ds(wid * 16, 16)], tb)
    cursor = tb[pl.ds(0, 16)][0]
    nu = (cursor + KR - 1) // KR

    _sc_phase2(y_hbm, lists_hbm, lbase, nu, acc, pbuf, idxb, dstu, rows,
               lsem, gsem, RING_R, KR)

    pltpu.sync_copy(acc.at[pl.ds(0, NPT)], out_hbm.at[pl.ds(lo, NPT)])


_SC_MESH = plsc.VectorSubcoreMesh(core_axis_name="c", subcore_axis_name="s")


def _ring_scratch(ring, k):
    return [
        pltpu.VMEM((ring, k), jnp.int32),
        pltpu.VMEM((ring, k), jnp.int32),
        pltpu.VMEM((ring, k), jnp.int32),
        pltpu.VMEM((ring, k, D), jnp.float32),
    ]


_sc_build = functools.partial(
    pl.kernel,
    out_type=(jax.ShapeDtypeStruct((NPAD, D), jnp.float32),
              jax.ShapeDtypeStruct((NW * LCAP,), jnp.int32),
              jax.ShapeDtypeStruct((NW * 16,), jnp.int32)),
    mesh=_SC_MESH,
    compiler_params=pltpu.CompilerParams(needs_layout_passes=False),
    scratch_types=[
        pltpu.VMEM((NPT + 1, D), jnp.float32),
        pltpu.VMEM((CHUNK,), jnp.int32),
        pltpu.VMEM((CHUNK,), jnp.int32),
        pltpu.VMEM((CHUNK,), jnp.int32),
        pltpu.VMEM((CHUNK,), jnp.int32),
        pltpu.VMEM((CPK_CAP,), jnp.int32),
        pltpu.VMEM((CPK_CAP,), jnp.int32),
    ] + _ring_scratch(RING_B, K) + [
        pltpu.VMEM((KR,), jnp.int32),
        pltpu.VMEM((16,), jnp.int32),
    ] + [pltpu.SemaphoreType.DMA] * (2 * RING_B + 4),
)(_sc_build_body)

_sc_reuse = functools.partial(
    pl.kernel,
    out_type=jax.ShapeDtypeStruct((NPAD, D), jnp.float32),
    mesh=_SC_MESH,
    compiler_params=pltpu.CompilerParams(needs_layout_passes=False),
    scratch_types=[
        pltpu.VMEM((NPT + 1, D), jnp.float32),
    ] + _ring_scratch(RING_R, KR) + [
        pltpu.VMEM((16,), jnp.int32),
    ] + [pltpu.SemaphoreType.DMA] * (2 * RING_R),
)(_sc_reuse_body)


def _shifts(v):
    z = jnp.zeros((v.shape[0], 1), v.dtype)
    vl = jnp.concatenate([z, v[:, :-1]], axis=1)
    vr = jnp.concatenate([v[:, 1:], z], axis=1)
    return vl, vr


def _bf(v):
    return v.astype(jnp.bfloat16).astype(jnp.float32)


def _conv3(v, w, b):
    vl, vr = _shifts(_bf(v))
    v = _bf(v)
    return _bf(w[0]) * vl + _bf(w[1]) * v + _bf(w[2]) * vr + b


def _conv3x2(v, a, w, b):
    vl, vr = _shifts(_bf(v))
    al, ar = _shifts(_bf(a))
    v = _bf(v)
    a = _bf(a)
    return (_bf(w[0]) * vl + _bf(w[1]) * v + _bf(w[2]) * vr
            + _bf(w[3]) * al + _bf(w[4]) * a + _bf(w[5]) * ar + b)


def _tc_pre_body(x_ref, w_ref, b_ref, y_ref):
    y_ref[...] = _conv3(x_ref[...], w_ref, b_ref[0])


def _tc_mid_body(x_ref, agg_ref, uw_ref, ub_ref, mw_ref, mb_ref,
                 h_ref, y_ref):
    a = agg_ref[...]
    a = jnp.where(jnp.isneginf(a), 0.0, a)
    h = jnp.maximum(_conv3x2(x_ref[...], a, uw_ref, ub_ref[0]), 0.0)
    h_ref[...] = h
    y_ref[...] = _conv3(h, mw_ref, mb_ref[0])


def _tc_final_body(h_ref, agg_ref, uw_ref, ub_ref, wt_ref, bp_ref, o_ref):
    a = agg_ref[...]
    a = jnp.where(jnp.isneginf(a), 0.0, a)
    h2 = jnp.maximum(_conv3x2(h_ref[...], a, uw_ref, ub_ref[0]), 0.0)
    m = jnp.max(h2, axis=1, keepdims=True)
    mb = m.astype(jnp.bfloat16).astype(jnp.float32)
    wb = wt_ref[...].astype(jnp.bfloat16).astype(jnp.float32)
    o_ref[...] = (jnp.sum(mb * wb, axis=0, keepdims=True)
                  + bp_ref[...])


_smem_spec = pl.BlockSpec(memory_space=pltpu.SMEM)
_vmem_spec = pl.BlockSpec(memory_space=pltpu.VMEM)

_tc_pre = pl.pallas_call(
    _tc_pre_body,
    out_shape=jax.ShapeDtypeStruct((N, D), jnp.float32),
    in_specs=[_vmem_spec, _smem_spec, _smem_spec],
    out_specs=_vmem_spec,
)

_tc_mid = pl.pallas_call(
    _tc_mid_body,
    out_shape=(jax.ShapeDtypeStruct((N, D), jnp.float32),
               jax.ShapeDtypeStruct((N, D), jnp.float32)),
    in_specs=[_vmem_spec, _vmem_spec, _smem_spec, _smem_spec,
              _smem_spec, _smem_spec],
    out_specs=(_vmem_spec, _vmem_spec),
)

_tc_final = pl.pallas_call(
    _tc_final_body,
    out_shape=jax.ShapeDtypeStruct((1, D), jnp.float32),
    in_specs=[_vmem_spec, _vmem_spec, _smem_spec, _smem_spec,
              _vmem_spec, _vmem_spec],
    out_specs=_vmem_spec,
)


def kernel(x, edge_index, mf_w0, mf_b0, uf_w0, uf_b0,
           mf_w1, mf_b1, uf_w1, uf_b1, W_out, b_out):
    src = edge_index[0]
    dst = edge_index[1]
    mw0 = mf_w0.reshape(3)
    uw0 = uf_w0.reshape(6)
    mw1 = mf_w1.reshape(3)
    uw1 = uf_w1.reshape(6)
    wt = jnp.pad(W_out.T, ((0, 0), (0, D - W_out.shape[0])))
    bp = jnp.pad(b_out, (0, D - b_out.shape[0]))[None, :]

    y0 = _tc_pre(x, mw0, mf_b0)
    agg0, lists, totals = _sc_build(y0, src, dst)
    h1, y1 = _tc_mid(x, agg0[:N], uw0, uf_b0, mw1, mf_b1)
    agg1 = _sc_reuse(y1, lists, totals)
    res = _tc_final(h1, agg1[:N], uw1, uf_b1, wt, bp)
    return res[:, :3]


# speedup vs baseline: 1.0544x; 1.0192x over previous
import functools

import jax
import jax.numpy as jnp
from jax import lax
from jax.experimental import pallas as pl
from jax.experimental.pallas import tpu as pltpu
from jax.experimental.pallas import tpu_sc as plsc

N = 10000
D = 128
E = 320000

NC = 2
NS = 16
NW = NC * NS
NPT = 320
NPAD = NW * NPT
CHUNK = 4000
NCHUNKS = E // CHUNK
K = 64
KR = 64
RING_B = 4
RING_R = 4
GL = 2
SPILL_BLK = 2048
TRASH = 4096
CPK_CAP = TRASH + 16
LCAP = E + 2048
DUMMY_PK = NPT


def _wid_lo():
    wid = lax.axis_index("s") * NC + lax.axis_index("c")
    return wid, wid * NPT


def _init_acc(acc):
    neg = jnp.full((16,), -jnp.inf, dtype=jnp.float32)

    def init_row(i, _):
        r = i // 8
        f = i % 8
        acc[r, pl.ds(f * 16, 16)] = neg
        return 0

    lax.fori_loop(0, (NPT + 1) * 8, init_row, 0, unroll=8)


def _sc_phase2(y_hbm, lists_hbm, lbase, nu, acc, pbuf, idxb, dstu, rows,
               lsem, gsem, ring, k):
    pb = [pbuf.at[j] for j in range(ring)]
    ib = [idxb.at[j] for j in range(ring)]
    db = [dstu.at[j] for j in range(ring)]
    rb = [rows.at[j] for j in range(ring)]
    ml = ring - 1

    def macro(ms, _):
        for j in range(ring):
            t = ms * ring + j

            @pl.when(t < nu)
            def _():
                pltpu.async_copy(lists_hbm.at[pl.ds(lbase + t * k, k)],
                                 pb[j], lsem[j])

            t2 = t - GL
            s2 = (j - GL) % ring

            @pl.when((t2 >= 0) & (t2 < nu))
            def _():
                pltpu.make_async_copy(
                    lists_hbm.at[pl.ds(lbase + t2 * k, k)],
                    pb[s2], lsem[s2]).wait()
                for r in range(k // 16):
                    v = pb[s2][pl.ds(r * 16, 16)]
                    ib[s2][pl.ds(r * 16, 16)] = v >> 9
                    db[s2][pl.ds(r * 16, 16)] = v & 511
                pltpu.async_copy(y_hbm.at[ib[s2]], rb[s2], gsem[s2])

            t3 = t - ml
            s3 = (j - ml) % ring

            @pl.when((t3 >= 0) & (t3 < nu))
            def _():
                pltpu.make_async_copy(y_hbm.at[ib[s3]], rb[s3],
                                      gsem[s3]).wait()

                def group(g, _):
                    dv = db[s3][pl.ds(g * 16, 16)]
                    for lane in range(16):
                        dl = dv[lane]
                        jr = g * 16 + lane
                        for f in range(8):
                            sl = pl.ds(f * 16, 16)
                            acc[dl, sl] = jnp.maximum(acc[dl, sl],
                                                      rb[s3][jr, sl])
                    return 0

                lax.fori_loop(0, k // 16, group, 0)
        return 0

    lax.fori_loop(0, (nu + ml + ring) // ring, macro, 0)


def _sc_build_body(y_hbm, src_hbm, dst_hbm,
                   out_hbm, lists_hbm, totals_hbm,
                   acc, srcb0, srcb1, dstb0, dstb1, cpk0, cpk1,
                   pbuf, idxb, dstu, rows, dbuf, tb,
                   ls0, ls1, ls2, ls3, gs0, gs1, gs2, gs3,
                   cs0, cs1, ss0, ss1):
    srcb = [srcb0, srcb1]
    dstb = [dstb0, dstb1]
    cpk = [cpk0, cpk1]
    lsem = [ls0, ls1, ls2, ls3]
    gsem = [gs0, gs1, gs2, gs3]
    csem = [cs0, cs1]
    ssem = [ss0, ss1]
    wid, lo = _wid_lo()
    _init_acc(acc)
    lbase = wid * LCAP

    dummy_pk = jnp.full((16,), DUMMY_PK, jnp.int32)
    lanes = lax.iota(jnp.int32, 16)
    for r in range(KR // 16):
        dbuf[pl.ds(r * 16, 16)] = dummy_pk

    def start_chunk(c, p):
        base = c * CHUNK
        pltpu.async_copy(src_hbm.at[pl.ds(base, CHUNK)], srcb[p],
                         csem[p])
        pltpu.async_copy(dst_hbm.at[pl.ds(base, CHUNK)], dstb[p],
                         csem[p])

    def wait_chunk(c, p):
        base = c * CHUNK
        pltpu.make_async_copy(src_hbm.at[pl.ds(base, CHUNK)], srcb[p],
                              csem[p]).wait()
        pltpu.make_async_copy(dst_hbm.at[pl.ds(base, CHUNK)], dstb[p],
                              csem[p]).wait()

    def drain_spill(p, nblk):
        def w(b, _):
            pltpu.make_async_copy(
                cpk[p].at[pl.ds(b * SPILL_BLK, SPILL_BLK)],
                lists_hbm.at[pl.ds(lbase, SPILL_BLK)],
                ssem[p]).wait()
            return 0
        lax.fori_loop(0, nblk, w, 0)

    start_chunk(0, 0)

    def chunk_pair(cc, carry):
        cursor, n0, n1 = carry
        nprev = [n0, n1]
        for p in range(2):
            c = cc * 2 + p
            drain_spill(p, nprev[p])
            wait_chunk(c, p)

            @pl.when(c + 1 < NCHUNKS)
            def _():
                start_chunk(c + 1, 1 - p)

            def filt(i, cnt):
                dv = dstb[p][pl.ds(i * 16, 16)]
                sv = srcb[p][pl.ds(i * 16, 16)]
                rel = dv - lo
                m = (rel >= 0) & (rel < NPT)
                incl = plsc.cumsum(jnp.where(m, 1, 0))
                pos = jnp.where(m, cnt + incl - 1, TRASH + lanes)
                plsc.store_scatter(cpk[p], [pos],
                                   (sv << 9) | (rel & 511))
                pc = plsc.all_reduce_population_count(m)
                return cnt + pc[0]

            cnt = lax.fori_loop(0, CHUNK // 16, filt, 0, unroll=4)
            cpk[p][pl.ds(cnt, 16)] = dummy_pk
            r8 = ((cnt + 7) // 8) * 8
            nblk = (r8 + SPILL_BLK - 1) // SPILL_BLK

            def spill(b, _):
                off = pl.multiple_of(lbase + cursor + b * SPILL_BLK, 8)
                pltpu.async_copy(
                    cpk[p].at[pl.ds(b * SPILL_BLK, SPILL_BLK)],
                    lists_hbm.at[pl.ds(off, SPILL_BLK)],
                    ssem[p])
                return 0

            lax.fori_loop(0, nblk, spill, 0)
            cursor = cursor + r8
            nprev[p] = nblk
        return (cursor, nprev[0], nprev[1])

    cursor, n0, n1 = lax.fori_loop(0, NCHUNKS // 2, chunk_pair, (0, 0, 0))
    drain_spill(0, n0)
    drain_spill(1, n1)

    pltpu.sync_copy(
        dbuf, lists_hbm.at[pl.ds(pl.multiple_of(lbase + cursor, 8), KR)])
    tb[pl.ds(0, 16)] = jnp.zeros((16,), jnp.int32) + cursor
    pltpu.sync_copy(tb, totals_hbm.at[pl.ds(wid * 16, 16)])

    nu = (cursor + K - 1) // K
    _sc_phase2(y_hbm, lists_hbm, lbase, nu, acc, pbuf, idxb, dstu, rows,
               lsem, gsem, RING_B, K)

    pltpu.sync_copy(acc.at[pl.ds(0, NPT)], out_hbm.at[pl.ds(lo, NPT)])


def _sc_reuse_body(y_hbm, lists_hbm, totals_hbm,
                   out_hbm,
                   acc, pbuf, idxb, dstu, rows, tb,
                   ls0, ls1, ls2, ls3, gs0, gs1, gs2, gs3):
    lsem = [ls0, ls1, ls2, ls3]
    gsem = [gs0, gs1, gs2, gs3]
    wid, lo = _wid_lo()
    _init_acc(acc)
    lbase = wid * LCAP

    pltpu.sync_copy(totals_hbm.at[pl.ds(wid * 16, 16)], tb)
    cursor = tb[pl.ds(0, 16)][0]
    nu = (cursor + KR - 1) // KR

    _sc_phase2(y_hbm, lists_hbm, lbase, nu, acc, pbuf, idxb, dstu, rows,
               lsem, gsem, RING_R, KR)

    pltpu.sync_copy(acc.at[pl.ds(0, NPT)], out_hbm.at[pl.ds(lo, NPT)])


_SC_MESH = plsc.VectorSubcoreMesh(core_axis_name="c", subcore_axis_name="s")


def _ring_scratch(ring, k):
    return [
        pltpu.VMEM((ring, k), jnp.int32),
        pltpu.VMEM((ring, k), jnp.int32),
        pltpu.VMEM((ring, k), jnp.int32),
        pltpu.VMEM((ring, k, D), jnp.float32),
    ]


_sc_build = functools.partial(
    pl.kernel,
    out_type=(jax.ShapeDtypeStruct((NPAD, D), jnp.float32),
              jax.ShapeDtypeStruct((NW * LCAP,), jnp.int32),
              jax.ShapeDtypeStruct((NW * 16,), jnp.int32)),
    mesh=_SC_MESH,
    compiler_params=pltpu.CompilerParams(needs_layout_passes=False),
    scratch_types=[
        pltpu.VMEM((NPT + 1, D), jnp.float32),
        pltpu.VMEM((CHUNK,), jnp.int32),
        pltpu.VMEM((CHUNK,), jnp.int32),
        pltpu.VMEM((CHUNK,), jnp.int32),
        pltpu.VMEM((CHUNK,), jnp.int32),
        pltpu.VMEM((CPK_CAP,), jnp.int32),
        pltpu.VMEM((CPK_CAP,), jnp.int32),
    ] + _ring_scratch(RING_B, K) + [
        pltpu.VMEM((KR,), jnp.int32),
        pltpu.VMEM((16,), jnp.int32),
    ] + [pltpu.SemaphoreType.DMA] * (2 * RING_B + 4),
)(_sc_build_body)

_sc_reuse = functools.partial(
    pl.kernel,
    out_type=jax.ShapeDtypeStruct((NPAD, D), jnp.float32),
    mesh=_SC_MESH,
    compiler_params=pltpu.CompilerParams(needs_layout_passes=False),
    scratch_types=[
        pltpu.VMEM((NPT + 1, D), jnp.float32),
    ] + _ring_scratch(RING_R, KR) + [
        pltpu.VMEM((16,), jnp.int32),
    ] + [pltpu.SemaphoreType.DMA] * (2 * RING_R),
)(_sc_reuse_body)


def _shifts(v):
    z = jnp.zeros((v.shape[0], 1), v.dtype)
    vl = jnp.concatenate([z, v[:, :-1]], axis=1)
    vr = jnp.concatenate([v[:, 1:], z], axis=1)
    return vl, vr


def _bf(v):
    return v.astype(jnp.bfloat16).astype(jnp.float32)


def _conv3(v, w, b):
    vl, vr = _shifts(_bf(v))
    v = _bf(v)
    return _bf(w[0]) * vl + _bf(w[1]) * v + _bf(w[2]) * vr + b


def _conv3x2(v, a, w, b):
    vl, vr = _shifts(_bf(v))
    al, ar = _shifts(_bf(a))
    v = _bf(v)
    a = _bf(a)
    return (_bf(w[0]) * vl + _bf(w[1]) * v + _bf(w[2]) * vr
            + _bf(w[3]) * al + _bf(w[4]) * a + _bf(w[5]) * ar + b)


def _tc_pre_body(x_ref, w_ref, b_ref, y_ref):
    y_ref[...] = _conv3(x_ref[...], w_ref, b_ref[0])


def _tc_mid_body(x_ref, agg_ref, uw_ref, ub_ref, mw_ref, mb_ref,
                 h_ref, y_ref):
    a = agg_ref[...]
    a = jnp.where(jnp.isneginf(a), 0.0, a)
    h = jnp.maximum(_conv3x2(x_ref[...], a, uw_ref, ub_ref[0]), 0.0)
    h_ref[...] = h
    y_ref[...] = _conv3(h, mw_ref, mb_ref[0])


def _tc_final_body(h_ref, agg_ref, uw_ref, ub_ref, wt_ref, bp_ref, o_ref):
    a = agg_ref[...]
    a = jnp.where(jnp.isneginf(a), 0.0, a)
    h2 = jnp.maximum(_conv3x2(h_ref[...], a, uw_ref, ub_ref[0]), 0.0)
    m = jnp.max(h2, axis=1, keepdims=True)
    mb = m.astype(jnp.bfloat16).astype(jnp.float32)
    wb = wt_ref[...].astype(jnp.bfloat16).astype(jnp.float32)
    o_ref[...] = (jnp.sum(mb * wb, axis=0, keepdims=True)
                  + bp_ref[...])


_smem_spec = pl.BlockSpec(memory_space=pltpu.SMEM)
_vmem_spec = pl.BlockSpec(memory_space=pltpu.VMEM)

_tc_pre = pl.pallas_call(
    _tc_pre_body,
    out_shape=jax.ShapeDtypeStruct((N, D), jnp.float32),
    in_specs=[_vmem_spec, _smem_spec, _smem_spec],
    out_specs=_vmem_spec,
)

_tc_mid = pl.pallas_call(
    _tc_mid_body,
    out_shape=(jax.ShapeDtypeStruct((N, D), jnp.float32),
               jax.ShapeDtypeStruct((N, D), jnp.float32)),
    in_specs=[_vmem_spec, _vmem_spec, _smem_spec, _smem_spec,
              _smem_spec, _smem_spec],
    out_specs=(_vmem_spec, _vmem_spec),
)

_tc_final = pl.pallas_call(
    _tc_final_body,
    out_shape=jax.ShapeDtypeStruct((1, D), jnp.float32),
    in_specs=[_vmem_spec, _vmem_spec, _smem_spec, _smem_spec,
              _vmem_spec, _vmem_spec],
    out_specs=_vmem_spec,
)


def kernel(x, edge_index, mf_w0, mf_b0, uf_w0, uf_b0,
           mf_w1, mf_b1, uf_w1, uf_b1, W_out, b_out):
    src = edge_index[0]
    dst = edge_index[1]
    mw0 = mf_w0.reshape(3)
    uw0 = uf_w0.reshape(6)
    mw1 = mf_w1.reshape(3)
    uw1 = uf_w1.reshape(6)
    wt = jnp.pad(W_out.T, ((0, 0), (0, D - W_out.shape[0])))
    bp = jnp.pad(b_out, (0, D - b_out.shape[0]))[None, :]

    y0 = _tc_pre(x, mw0, mf_b0)
    agg0, lists, totals = _sc_build(y0, src, dst)
    h1, y1 = _tc_mid(x, agg0[:N], uw0, uf_b0, mw1, mf_b1)
    agg1 = _sc_reuse(y1, lists, totals)
    res = _tc_final(h1, agg1[:N], uw1, uf_b1, wt, bp)
    return res[:, :3]


# speedup vs baseline: 1.1423x; 1.0834x over previous
import functools

import jax
import jax.numpy as jnp
from jax import lax
from jax.experimental import pallas as pl
from jax.experimental.pallas import tpu as pltpu
from jax.experimental.pallas import tpu_sc as plsc

N = 10000
D = 128
E = 320000

NC = 2
NS = 16
NW = NC * NS
NPT = 320
NPAD = NW * NPT
CHUNK = 8000
NCHUNKS = E // CHUNK
K = 64
KR = 64
RING_B = 4
RING_R = 4
GL = 2
SPILL_BLK = 2048
TRASH = 8192
CPK_CAP = TRASH + 16
LCAP = E + 2048
DUMMY_PK = NPT


def _wid_lo():
    wid = lax.axis_index("s") * NC + lax.axis_index("c")
    return wid, wid * NPT


def _init_acc(acc):
    neg = jnp.full((16,), -jnp.inf, dtype=jnp.float32)

    def init_row(i, _):
        r = i // 8
        f = i % 8
        acc[r, pl.ds(f * 16, 16)] = neg
        return 0

    lax.fori_loop(0, (NPT + 1) * 8, init_row, 0, unroll=8)


def _sc_phase2(y_hbm, lists_hbm, lbase, nu, acc, pbuf, idxb, dstu, rows,
               lsem, gsem, ring, k):
    pb = [pbuf.at[j] for j in range(ring)]
    ib = [idxb.at[j] for j in range(ring)]
    db = [dstu.at[j] for j in range(ring)]
    rb = [rows.at[j] for j in range(ring)]
    ml = ring - 1

    def macro(ms, _):
        for j in range(ring):
            t = ms * ring + j

            @pl.when(t < nu)
            def _():
                pltpu.async_copy(lists_hbm.at[pl.ds(lbase + t * k, k)],
                                 pb[j], lsem[j])

            t2 = t - GL
            s2 = (j - GL) % ring

            @pl.when((t2 >= 0) & (t2 < nu))
            def _():
                pltpu.make_async_copy(
                    lists_hbm.at[pl.ds(lbase + t2 * k, k)],
                    pb[s2], lsem[s2]).wait()
                for r in range(k // 16):
                    v = pb[s2][pl.ds(r * 16, 16)]
                    ib[s2][pl.ds(r * 16, 16)] = v >> 9
                    db[s2][pl.ds(r * 16, 16)] = v & 511
                pltpu.async_copy(y_hbm.at[ib[s2]], rb[s2], gsem[s2])

            t3 = t - ml
            s3 = (j - ml) % ring

            @pl.when((t3 >= 0) & (t3 < nu))
            def _():
                pltpu.make_async_copy(y_hbm.at[ib[s3]], rb[s3],
                                      gsem[s3]).wait()

                def group(g, _):
                    dv = db[s3][pl.ds(g * 16, 16)]
                    for lane in range(16):
                        dl = dv[lane]
                        jr = g * 16 + lane
                        for f in range(8):
                            sl = pl.ds(f * 16, 16)
                            acc[dl, sl] = jnp.maximum(acc[dl, sl],
                                                      rb[s3][jr, sl])
                    return 0

                lax.fori_loop(0, k // 16, group, 0)
        return 0

    lax.fori_loop(0, (nu + ml + ring) // ring, macro, 0)


def _sc_build_body(y_hbm, src_hbm, dst_hbm,
                   out_hbm, lists_hbm, totals_hbm,
                   acc, srcb0, srcb1, dstb0, dstb1, cpk0, cpk1,
                   pbuf, idxb, dstu, rows, dbuf, tb,
                   ls0, ls1, ls2, ls3, gs0, gs1, gs2, gs3,
                   cs0, cs1, ss0, ss1):
    srcb = [srcb0, srcb1]
    dstb = [dstb0, dstb1]
    cpk = [cpk0, cpk1]
    lsem = [ls0, ls1, ls2, ls3]
    gsem = [gs0, gs1, gs2, gs3]
    csem = [cs0, cs1]
    ssem = [ss0, ss1]
    wid, lo = _wid_lo()
    _init_acc(acc)
    lbase = wid * LCAP

    dummy_pk = jnp.full((16,), DUMMY_PK, jnp.int32)
    lanes = lax.iota(jnp.int32, 16)
    for r in range(KR // 16):
        dbuf[pl.ds(r * 16, 16)] = dummy_pk

    def start_chunk(c, p):
        base = c * CHUNK
        pltpu.async_copy(src_hbm.at[pl.ds(base, CHUNK)], srcb[p],
                         csem[p])
        pltpu.async_copy(dst_hbm.at[pl.ds(base, CHUNK)], dstb[p],
                         csem[p])

    def wait_chunk(c, p):
        base = c * CHUNK
        pltpu.make_async_copy(src_hbm.at[pl.ds(base, CHUNK)], srcb[p],
                              csem[p]).wait()
        pltpu.make_async_copy(dst_hbm.at[pl.ds(base, CHUNK)], dstb[p],
                              csem[p]).wait()

    def drain_spill(p, nblk):
        def w(b, _):
            pltpu.make_async_copy(
                cpk[p].at[pl.ds(b * SPILL_BLK, SPILL_BLK)],
                lists_hbm.at[pl.ds(lbase, SPILL_BLK)],
                ssem[p]).wait()
            return 0
        lax.fori_loop(0, nblk, w, 0)

    start_chunk(0, 0)

    def chunk_pair(cc, carry):
        cursor, n0, n1 = carry
        nprev = [n0, n1]
        for p in range(2):
            c = cc * 2 + p
            drain_spill(p, nprev[p])
            wait_chunk(c, p)

            @pl.when(c + 1 < NCHUNKS)
            def _():
                start_chunk(c + 1, 1 - p)

            def filt(i, cnt):
                dv = dstb[p][pl.ds(i * 16, 16)]
                sv = srcb[p][pl.ds(i * 16, 16)]
                rel = dv - lo
                m = (rel >= 0) & (rel < NPT)
                incl = plsc.cumsum(jnp.where(m, 1, 0))
                pos = jnp.where(m, cnt + incl - 1, TRASH + lanes)
                plsc.store_scatter(cpk[p], [pos],
                                   (sv << 9) | (rel & 511))
                pc = plsc.all_reduce_population_count(m)
                return cnt + pc[0]

            cnt = lax.fori_loop(0, CHUNK // 16, filt, 0, unroll=4)
            cpk[p][pl.ds(cnt, 16)] = dummy_pk
            r8 = ((cnt + 7) // 8) * 8
            nblk = (r8 + SPILL_BLK - 1) // SPILL_BLK

            def spill(b, _):
                off = pl.multiple_of(lbase + cursor + b * SPILL_BLK, 8)
                pltpu.async_copy(
                    cpk[p].at[pl.ds(b * SPILL_BLK, SPILL_BLK)],
                    lists_hbm.at[pl.ds(off, SPILL_BLK)],
                    ssem[p])
                return 0

            lax.fori_loop(0, nblk, spill, 0)
            cursor = cursor + r8
            nprev[p] = nblk
        return (cursor, nprev[0], nprev[1])

    cursor, n0, n1 = lax.fori_loop(0, NCHUNKS // 2, chunk_pair, (0, 0, 0))
    drain_spill(0, n0)
    drain_spill(1, n1)

    pltpu.sync_copy(
        dbuf, lists_hbm.at[pl.ds(pl.multiple_of(lbase + cursor, 8), KR)])
    tb[pl.ds(0, 16)] = jnp.zeros((16,), jnp.int32) + cursor
    pltpu.sync_copy(tb, totals_hbm.at[pl.ds(wid * 16, 16)])

    nu = (cursor + K - 1) // K
    _sc_phase2(y_hbm, lists_hbm, lbase, nu, acc, pbuf, idxb, dstu, rows,
               lsem, gsem, RING_B, K)

    pltpu.sync_copy(acc.at[pl.ds(0, NPT)], out_hbm.at[pl.ds(lo, NPT)])


def _sc_reuse_body(y_hbm, lists_hbm, totals_hbm,
                   out_hbm,
                   acc, pbuf, idxb, dstu, rows, tb,
                   ls0, ls1, ls2, ls3, gs0, gs1, gs2, gs3):
    lsem = [ls0, ls1, ls2, ls3]
    gsem = [gs0, gs1, gs2, gs3]
    wid, lo = _wid_lo()
    _init_acc(acc)
    lbase = wid * LCAP

    pltpu.sync_copy(totals_hbm.at[pl.ds(wid * 16, 16)], tb)
    cursor = tb[pl.ds(0, 16)][0]
    nu = (cursor + KR - 1) // KR

    _sc_phase2(y_hbm, lists_hbm, lbase, nu, acc, pbuf, idxb, dstu, rows,
               lsem, gsem, RING_R, KR)

    pltpu.sync_copy(acc.at[pl.ds(0, NPT)], out_hbm.at[pl.ds(lo, NPT)])


_SC_MESH = plsc.VectorSubcoreMesh(core_axis_name="c", subcore_axis_name="s")


def _ring_scratch(ring, k):
    return [
        pltpu.VMEM((ring, k), jnp.int32),
        pltpu.VMEM((ring, k), jnp.int32),
        pltpu.VMEM((ring, k), jnp.int32),
        pltpu.VMEM((ring, k, D), jnp.float32),
    ]


_sc_build = functools.partial(
    pl.kernel,
    out_type=(jax.ShapeDtypeStruct((NPAD, D), jnp.float32),
              jax.ShapeDtypeStruct((NW * LCAP,), jnp.int32),
              jax.ShapeDtypeStruct((NW * 16,), jnp.int32)),
    mesh=_SC_MESH,
    compiler_params=pltpu.CompilerParams(needs_layout_passes=False),
    scratch_types=[
        pltpu.VMEM((NPT + 1, D), jnp.float32),
        pltpu.VMEM((CHUNK,), jnp.int32),
        pltpu.VMEM((CHUNK,), jnp.int32),
        pltpu.VMEM((CHUNK,), jnp.int32),
        pltpu.VMEM((CHUNK,), jnp.int32),
        pltpu.VMEM((CPK_CAP,), jnp.int32),
        pltpu.VMEM((CPK_CAP,), jnp.int32),
    ] + _ring_scratch(RING_B, K) + [
        pltpu.VMEM((KR,), jnp.int32),
        pltpu.VMEM((16,), jnp.int32),
    ] + [pltpu.SemaphoreType.DMA] * (2 * RING_B + 4),
)(_sc_build_body)

_sc_reuse = functools.partial(
    pl.kernel,
    out_type=jax.ShapeDtypeStruct((NPAD, D), jnp.float32),
    mesh=_SC_MESH,
    compiler_params=pltpu.CompilerParams(needs_layout_passes=False),
    scratch_types=[
        pltpu.VMEM((NPT + 1, D), jnp.float32),
    ] + _ring_scratch(RING_R, KR) + [
        pltpu.VMEM((16,), jnp.int32),
    ] + [pltpu.SemaphoreType.DMA] * (2 * RING_R),
)(_sc_reuse_body)


def _shifts(v):
    z = jnp.zeros((v.shape[0], 1), v.dtype)
    vl = jnp.concatenate([z, v[:, :-1]], axis=1)
    vr = jnp.concatenate([v[:, 1:], z], axis=1)
    return vl, vr


def _bf(v):
    return v.astype(jnp.bfloat16).astype(jnp.float32)


def _conv3(v, w, b):
    vl, vr = _shifts(_bf(v))
    v = _bf(v)
    return _bf(w[0]) * vl + _bf(w[1]) * v + _bf(w[2]) * vr + b


def _conv3x2(v, a, w, b):
    vl, vr = _shifts(_bf(v))
    al, ar = _shifts(_bf(a))
    v = _bf(v)
    a = _bf(a)
    return (_bf(w[0]) * vl + _bf(w[1]) * v + _bf(w[2]) * vr
            + _bf(w[3]) * al + _bf(w[4]) * a + _bf(w[5]) * ar + b)


def _tc_pre_body(x_ref, w_ref, b_ref, y_ref):
    y_ref[...] = _conv3(x_ref[...], w_ref, b_ref[0])


def _tc_mid_body(x_ref, agg_ref, uw_ref, ub_ref, mw_ref, mb_ref,
                 h_ref, y_ref):
    a = agg_ref[...]
    a = jnp.where(jnp.isneginf(a), 0.0, a)
    h = jnp.maximum(_conv3x2(x_ref[...], a, uw_ref, ub_ref[0]), 0.0)
    h_ref[...] = h
    y_ref[...] = _conv3(h, mw_ref, mb_ref[0])


def _tc_final_body(h_ref, agg_ref, uw_ref, ub_ref, wt_ref, bp_ref, o_ref):
    a = agg_ref[...]
    a = jnp.where(jnp.isneginf(a), 0.0, a)
    h2 = jnp.maximum(_conv3x2(h_ref[...], a, uw_ref, ub_ref[0]), 0.0)
    m = jnp.max(h2, axis=1, keepdims=True)
    mb = m.astype(jnp.bfloat16).astype(jnp.float32)
    wb = wt_ref[...].astype(jnp.bfloat16).astype(jnp.float32)
    o_ref[...] = (jnp.sum(mb * wb, axis=0, keepdims=True)
                  + bp_ref[...])


_smem_spec = pl.BlockSpec(memory_space=pltpu.SMEM)
_vmem_spec = pl.BlockSpec(memory_space=pltpu.VMEM)

_tc_pre = pl.pallas_call(
    _tc_pre_body,
    out_shape=jax.ShapeDtypeStruct((N, D), jnp.float32),
    in_specs=[_vmem_spec, _smem_spec, _smem_spec],
    out_specs=_vmem_spec,
)

_tc_mid = pl.pallas_call(
    _tc_mid_body,
    out_shape=(jax.ShapeDtypeStruct((N, D), jnp.float32),
               jax.ShapeDtypeStruct((N, D), jnp.float32)),
    in_specs=[_vmem_spec, _vmem_spec, _smem_spec, _smem_spec,
              _smem_spec, _smem_spec],
    out_specs=(_vmem_spec, _vmem_spec),
)

_tc_final = pl.pallas_call(
    _tc_final_body,
    out_shape=jax.ShapeDtypeStruct((1, D), jnp.float32),
    in_specs=[_vmem_spec, _vmem_spec, _smem_spec, _smem_spec,
              _vmem_spec, _vmem_spec],
    out_specs=_vmem_spec,
)


def kernel(x, edge_index, mf_w0, mf_b0, uf_w0, uf_b0,
           mf_w1, mf_b1, uf_w1, uf_b1, W_out, b_out):
    src = edge_index[0]
    dst = edge_index[1]
    mw0 = mf_w0.reshape(3)
    uw0 = uf_w0.reshape(6)
    mw1 = mf_w1.reshape(3)
    uw1 = uf_w1.reshape(6)
    wt = jnp.pad(W_out.T, ((0, 0), (0, D - W_out.shape[0])))
    bp = jnp.pad(b_out, (0, D - b_out.shape[0]))[None, :]

    y0 = _tc_pre(x, mw0, mf_b0)
    agg0, lists, totals = _sc_build(y0, src, dst)
    h1, y1 = _tc_mid(x, agg0[:N], uw0, uf_b0, mw1, mf_b1)
    agg1 = _sc_reuse(y1, lists, totals)
    res = _tc_final(h1, agg1[:N], uw1, uf_b1, wt, bp)
    return res[:, :3]
